# baseline (speedup 1.0000x reference)
import functools

import jax
import jax.numpy as jnp
from jax import lax
from jax.experimental import pallas as pl
from jax.experimental.pallas import tpu as pltpu
from jax.experimental.pallas import tpu_sc as plsc


def kernel(x, table):
    bs, seq_len = x.shape
    num_rows, emb = table.shape

    info = plsc.get_sparse_core_info()
    nw = info.num_cores * info.num_subcores
    rows_per = seq_len // nw

    mesh = plsc.VectorSubcoreMesh(core_axis_name="c", subcore_axis_name="s")

    @functools.partial(
        pl.kernel,
        mesh=mesh,
        out_type=jax.ShapeDtypeStruct((seq_len, emb), table.dtype),
    )
    def positional_lookup(table_hbm, out_hbm):
        wid = lax.axis_index("s") * info.num_cores + lax.axis_index("c")
        base = wid * rows_per
        src = table_hbm.at[pl.ds(base, rows_per)]
        dst = out_hbm.at[pl.ds(base, rows_per)]
        pltpu.sync_copy(src, dst)

    return positional_lookup(table)


# speedup vs baseline: 23.2111x; 23.2111x over previous
import functools

import jax
import jax.numpy as jnp
from jax import lax
from jax.experimental import pallas as pl
from jax.experimental.pallas import tpu as pltpu
from jax.experimental.pallas import tpu_sc as plsc


def kernel(x, table):
    bs, seq_len = x.shape
    num_rows, emb = table.shape

    info = plsc.get_sparse_core_info()
    nw = info.num_cores * info.num_subcores
    rows_per = seq_len // nw

    mesh = plsc.VectorSubcoreMesh(core_axis_name="c", subcore_axis_name="s")

    chunk = 32
    nchunks = rows_per // chunk

    @functools.partial(
        pl.kernel,
        mesh=mesh,
        out_type=jax.ShapeDtypeStruct((seq_len, emb), table.dtype),
        scratch_types=[
            pltpu.VMEM((2, chunk, emb), table.dtype),
            pltpu.SemaphoreType.DMA,
            pltpu.SemaphoreType.DMA,
            pltpu.SemaphoreType.DMA,
            pltpu.SemaphoreType.DMA,
        ],
    )
    def positional_lookup(table_hbm, out_hbm, buf, sin0, sin1, sout0, sout1):
        wid = lax.axis_index("s") * info.num_cores + lax.axis_index("c")
        base = wid * rows_per
        sin = (sin0, sin1)
        sout = (sout0, sout1)

        def in_copy(g, b):
            return pltpu.make_async_copy(
                table_hbm.at[pl.ds(base + g * chunk, chunk)], buf.at[b], sin[b]
            )

        def out_copy(g, b):
            return pltpu.make_async_copy(
                buf.at[b], out_hbm.at[pl.ds(base + g * chunk, chunk)], sout[b]
            )

        in_copy(0, 0).start()
        for g in range(nchunks):
            b = g & 1
            in_copy(g, b).wait()
            if g + 1 < nchunks:
                if g >= 1:
                    out_copy(g - 1, 1 - b).wait()
                in_copy(g + 1, 1 - b).start()
            out_copy(g, b).start()
        out_copy(nchunks - 1, (nchunks - 1) & 1).wait()

    return positional_lookup(table)
